# Initial kernel scaffold; baseline (speedup 1.0000x reference)
import jax
import jax.numpy as jnp
from jax.experimental import pallas as pl


def kernel(h, edge_index, W1, b1, W2, b2, W3, b3):
    raise NotImplementedError("write your pallas kernel here")


# speedup vs baseline: 4.6924x; 4.6924x over previous
import functools

import numpy as np
import jax
import jax.numpy as jnp
from jax import lax
from jax.experimental import pallas as pl
from jax.experimental.pallas import tpu as pltpu
from jax.experimental.pallas import tpu_sc as plsc

_N = 10000
_E = 320000
_D = 128
_NT = 32
_NB = 64
_NPB = 160
_CPT = 2 * _NPB
_NPAD = _NB * _NPB
_DUMMY = _NPB
_ROWS = _NPB + 1
_PB = 4000
_FB = 512
_OUTCAP = _FB + _PB + 32
_EC = 64
_BIN_CAP = _E + 256

_BN = 1000

_DEG_HIST = np.array([0, 1200, 2400, 3000, 2000, 900, 400, 80, 20], dtype=np.float64)
_DELTA = float((_DEG_HIST * np.log(np.arange(len(_DEG_HIST)) + 1.0)).sum() / _DEG_HIST.sum())

_mesh = plsc.VectorSubcoreMesh(core_axis_name="c", subcore_axis_name="s")
_scp = pltpu.CompilerParams(needs_layout_passes=False)


def _wid():
    return lax.axis_index("s") * 2 + lax.axis_index("c")


@functools.partial(
    pl.kernel,
    mesh=_mesh,
    compiler_params=_scp,
    out_type=[
        jax.ShapeDtypeStruct((_NB * _BIN_CAP,), jnp.int32),
        jax.ShapeDtypeStruct((_NB * 16,), jnp.int32),
    ],
    scratch_types=[
        pltpu.VMEM((_PB,), jnp.int32),
        pltpu.VMEM((_PB,), jnp.int32),
        pltpu.VMEM((_OUTCAP,), jnp.int32),
        pltpu.VMEM((_OUTCAP,), jnp.int32),
        pltpu.VMEM((16,), jnp.int32),
    ],
)
def _prep(esrc, edst, binned, counts, srcb, dstb, outb0, outb1, cntb):
    w = _wid()
    lo = w * _CPT
    dummy_vec = jnp.full((16,), _DUMMY, jnp.int32)
    outbs = (outb0, outb1)

    def flush(outb, bin_id, ptr, total, blk):
        off = pl.multiple_of(bin_id * _BIN_CAP + total, 128)
        pltpu.sync_copy(outb.at[pl.ds(0, blk)], binned.at[pl.ds(off, blk)])
        nrem = ptr - blk

        def shift_body(i, _):
            outb[pl.ds(i * 16, 16)] = outb[pl.ds(blk + i * 16, 16)]
            return 0

        lax.fori_loop(0, (nrem + 15) // 16, shift_body, 0)
        return ptr - blk, total + blk

    def chunk_body(c, state):
        ptr0, tot0, ptr1, tot1 = state
        pltpu.sync_copy(esrc.at[pl.ds(c * _PB, _PB)], srcb)
        pltpu.sync_copy(edst.at[pl.ds(c * _PB, _PB)], dstb)

        def vec_body(j, st):
            ptr0, ptr1 = st
            groups = []
            for u in range(2):
                sl = pl.ds((2 * j + u) * 16, 16)
                s = srcb[sl]
                dl = dstb[sl] - lo
                sp = jnp.left_shift(s, 8)
                for b in range(2):
                    dlb = dl - b * _NPB
                    m = (dlb >= 0) & (dlb < _NPB)
                    p = sp | jnp.where(m, dlb, 0)
                    cs = plsc.cumsum(m.astype(jnp.int32))
                    groups.append((b, m, p, cs))
            for b, m, p, cs in groups:
                ptr = ptr0 if b == 0 else ptr1
                pos = jnp.where(m, ptr + cs - 1, _OUTCAP - 1)
                plsc.store_scatter(outbs[b], [pos], p)
                if b == 0:
                    ptr0 = ptr0 + cs[15]
                else:
                    ptr1 = ptr1 + cs[15]
            return ptr0, ptr1

        ptr0, ptr1 = lax.fori_loop(0, _PB // 32, vec_body, (ptr0, ptr1))
        ptr0, tot0 = lax.while_loop(
            lambda st: st[0] >= _FB,
            lambda st: flush(outb0, 2 * w, st[0], st[1], _FB), (ptr0, tot0))
        ptr1, tot1 = lax.while_loop(
            lambda st: st[0] >= _FB,
            lambda st: flush(outb1, 2 * w + 1, st[0], st[1], _FB), (ptr1, tot1))
        return ptr0, tot0, ptr1, tot1

    state = lax.fori_loop(0, _E // _PB, chunk_body, (0, 0, 0, 0))

    for b in range(2):
        outb = outbs[b]
        ptr, total = state[2 * b], state[2 * b + 1]
        outb[pl.ds(ptr, 16)] = dummy_vec
        ptr = ((ptr + 16) // 16) * 16

        def pad_body(p):
            outb[pl.ds(p, 16)] = dummy_vec
            return p + 16

        ptr = lax.while_loop(lambda p: lax.rem(p, 2 * _EC) != 0, pad_body, ptr)
        ptr, total = lax.while_loop(
            lambda st: st[0] > 0,
            lambda st: flush(outb, 2 * w + b, st[0], st[1], 2 * _EC), (ptr, total))
        cntb[...] = jnp.full((16,), 0, jnp.int32) + total
        pltpu.sync_copy(
            cntb, counts.at[pl.ds(pl.multiple_of((2 * w + b) * 16, 16), 16)])


_NQ = _D // 16


def _make_agg(with_deg):
  out_type = [
      jax.ShapeDtypeStruct((_NPAD, _D), jnp.float32),
      jax.ShapeDtypeStruct((_NPAD, _D), jnp.float32),
      jax.ShapeDtypeStruct((_NPAD, _D), jnp.float32),
      jax.ShapeDtypeStruct((_NPAD, _D), jnp.float32),
  ]
  if with_deg:
      out_type.append(
          jax.ShapeDtypeStruct((_NPAD * 16,), jnp.float32))

  @functools.partial(
      pl.kernel,
      mesh=_mesh,
      compiler_params=_scp,
      out_type=out_type,
      scratch_types=(
          [pltpu.VMEM((_ROWS * 16,), jnp.float32) for _ in range(4 * _NQ + 1)]
          + [
              pltpu.VMEM((_EC,), jnp.int32),
              pltpu.VMEM((_EC,), jnp.int32),
              pltpu.VMEM((_EC,), jnp.int32),
              pltpu.VMEM((_EC, _D), jnp.float32),
              pltpu.VMEM((_EC,), jnp.int32),
              pltpu.VMEM((_EC,), jnp.int32),
              pltpu.VMEM((_EC,), jnp.int32),
              pltpu.VMEM((_EC, _D), jnp.float32),
              pltpu.VMEM((16,), jnp.int32),
              pltpu.SemaphoreType.DMA,
              pltpu.SemaphoreType.DMA,
          ]
      ),
  )
  def _agg(h, binned, counts, ssum, ssq, smx, smn, *rest):
    if with_deg:
        deg = rest[0]
        scratch = rest[1:]
    else:
        deg = None
        scratch = rest
    accs = scratch[:4 * _NQ]
    asum_q = accs[0:_NQ]
    asq_q = accs[_NQ:2 * _NQ]
    amx_q = accs[2 * _NQ:3 * _NQ]
    amn_q = accs[3 * _NQ:4 * _NQ]
    adeg = scratch[4 * _NQ]
    bufA = scratch[4 * _NQ + 1:4 * _NQ + 5]
    bufB = scratch[4 * _NQ + 5:4 * _NQ + 9]
    cntb = scratch[4 * _NQ + 9]
    semA = scratch[4 * _NQ + 10]
    semB = scratch[4 * _NQ + 11]

    w = _wid()
    zero16 = jnp.zeros((16,), jnp.float32)
    ones16 = jnp.ones((16,), jnp.float32)
    ninf16 = jnp.full((16,), -jnp.inf, jnp.float32)
    pinf16 = jnp.full((16,), jnp.inf, jnp.float32)

    for ph in range(2):
        bin_id = 2 * w + ph
        lo = pl.multiple_of(bin_id * _NPB, 32)
        base = pl.multiple_of(bin_id * _BIN_CAP, 128)
        pltpu.sync_copy(
            counts.at[pl.ds(pl.multiple_of(bin_id * 16, 16), 16)], cntb)
        nch = cntb[...][0] // _EC
        npair = nch // 2

        def init_body(r, _):
            sl = pl.ds(r * 16, 16)
            for q in range(_NQ):
                asum_q[q][sl] = zero16
                asq_q[q][sl] = zero16
                amx_q[q][sl] = ninf16
                amn_q[q][sl] = pinf16
            if with_deg:
                adeg[sl] = zero16
            return 0

        lax.fori_loop(0, _ROWS, init_body, 0)

        def load_and_start(buf, sem, ch):
            pbuf, gidx, dloc, rows = buf
            off = base + ch * _EC
            pltpu.sync_copy(binned.at[pl.ds(off, _EC)], pbuf)
            for j in range(_EC // 16):
                sl = pl.ds(j * 16, 16)
                p = pbuf[sl]
                dloc[sl] = p & 255
                gidx[sl] = jnp.right_shift(p, 8)
            pltpu.async_copy(h.at[gidx], rows, sem)

        def wait_gather(buf, sem):
            pbuf, gidx, dloc, rows = buf
            pltpu.make_async_copy(h.at[gidx], rows, sem).wait()

        def process(buf):
            pbuf, gidx, dloc, rows = buf

            def group_body(j, _):
                dvec = dloc[pl.ds(j * 16, 16)]
                for k in range(16):
                    e = j * 16 + k
                    ds = pl.ds(dvec[k] * 16, 16)
                    vs = [rows[e, pl.ds(q * 16, 16)] for q in range(_NQ)]
                    mxs = [amx_q[q][ds] for q in range(_NQ)]
                    mns = [amn_q[q][ds] for q in range(_NQ)]
                    for q in range(_NQ):
                        plsc.addupdate(asum_q[q].at[ds], vs[q])
                    for q in range(_NQ):
                        plsc.addupdate(asq_q[q].at[ds], vs[q] * vs[q])
                    for q in range(_NQ):
                        amx_q[q][ds] = jnp.maximum(mxs[q], vs[q])
                    for q in range(_NQ):
                        amn_q[q][ds] = jnp.minimum(mns[q], vs[q])
                    if with_deg:
                        plsc.addupdate(adeg.at[ds], ones16)
                return 0

            lax.fori_loop(0, _EC // 16, group_body, 0)

        load_and_start(bufA, semA, 0)

        def pair_body(i, _):
            load_and_start(bufB, semB, 2 * i + 1)
            wait_gather(bufA, semA)
            process(bufA)
            load_and_start(bufA, semA, jnp.minimum(2 * i + 2, nch - 1))
            wait_gather(bufB, semB)
            process(bufB)
            return 0

        lax.fori_loop(0, npair, pair_body, 0)
        wait_gather(bufA, semA)

        rowsA = bufA[3]
        for acc_qs, out in ((asum_q, ssum), (asq_q, ssq), (amx_q, smx),
                            (amn_q, smn)):
            for r0, nr in ((0, _EC), (_EC, _EC), (2 * _EC, _NPB - 2 * _EC)):
                def merge_body(r, _):
                    for q in range(_NQ):
                        rowsA[r, pl.ds(q * 16, 16)] = acc_qs[q][pl.ds((r0 + r) * 16, 16)]
                    return 0

                lax.fori_loop(0, nr, merge_body, 0)
                pltpu.sync_copy(rowsA.at[pl.ds(0, nr)],
                                out.at[pl.ds(lo + r0, nr)])
        if with_deg:
            lo16 = pl.multiple_of(bin_id * _NPB * 16, 128)
            pltpu.sync_copy(adeg.at[pl.ds(0, _NPB * 16)],
                            deg.at[pl.ds(lo16, _NPB * 16)])

  return _agg


_agg_deg = _make_agg(True)
_agg_nodeg = _make_agg(False)


def _round_body(h_ref, out_ref):
    out_ref[...] = jnp.round(h_ref[...] * 100.0) / 100.0


def _round_call(h):
    return pl.pallas_call(
        _round_body,
        grid=(_N // _BN,),
        in_specs=[pl.BlockSpec((_BN, _D), lambda i: (i, 0))],
        out_specs=pl.BlockSpec((_BN, _D), lambda i: (i, 0)),
        out_shape=jax.ShapeDtypeStruct((_N, _D), jnp.float32),
    )(h)


def _dense_body(ssum_ref, ssq_ref, smx_ref, smn_ref, deg_ref, W_ref, b_ref,
                out_ref, g_ref):
    deg = deg_ref[...][:, 0:1]
    degc = jnp.maximum(deg, 1.0)
    mean = ssum_ref[...] / degc
    sqmean = ssq_ref[...] / degc
    std = jnp.sqrt(jnp.maximum(sqmean - mean * mean, 0.0) + 1e-5)
    pos = deg > 0.0
    mx = jnp.where(pos, smx_ref[...], 0.0)
    mn = jnp.where(pos, smn_ref[...], 0.0)
    aggs = jnp.concatenate([mean, mx, mn, std], axis=1)
    logd = jnp.log(deg + 1.0)
    amp = logd / _DELTA
    att = _DELTA / jnp.maximum(logd, 1e-5)
    W = W_ref[...]
    acc = jnp.dot(aggs, W[0:512], preferred_element_type=jnp.float32,
                  precision=jax.lax.Precision.HIGHEST)
    acc = acc + amp * jnp.dot(aggs, W[512:1024], preferred_element_type=jnp.float32,
                              precision=jax.lax.Precision.HIGHEST)
    acc = acc + att * jnp.dot(aggs, W[1024:1536], preferred_element_type=jnp.float32,
                              precision=jax.lax.Precision.HIGHEST)
    h = jnp.maximum(acc + b_ref[...], 0.0)
    out_ref[...] = h

    @pl.when(pl.program_id(0) == 0)
    def _():
        g_ref[...] = jnp.zeros_like(g_ref)

    g_ref[...] += jnp.sum(h, axis=0, keepdims=True)


def _dense_layer(ssum, ssq, smx, smn, deg, W, b):
    return pl.pallas_call(
        _dense_body,
        grid=(_N // _BN,),
        in_specs=[
            pl.BlockSpec((_BN, _D), lambda i: (i, 0)),
            pl.BlockSpec((_BN, _D), lambda i: (i, 0)),
            pl.BlockSpec((_BN, _D), lambda i: (i, 0)),
            pl.BlockSpec((_BN, _D), lambda i: (i, 0)),
            pl.BlockSpec((_BN, 16), lambda i: (i, 0)),
            pl.BlockSpec((512 * 3, _D), lambda i: (0, 0)),
            pl.BlockSpec((1, _D), lambda i: (0, 0)),
        ],
        out_specs=[
            pl.BlockSpec((_BN, _D), lambda i: (i, 0)),
            pl.BlockSpec((1, _D), lambda i: (0, 0)),
        ],
        out_shape=[
            jax.ShapeDtypeStruct((_N, _D), jnp.float32),
            jax.ShapeDtypeStruct((1, _D), jnp.float32),
        ],
    )(ssum, ssq, smx, smn, deg, W, b.reshape(1, _D))


def kernel(h, edge_index, W1, b1, W2, b2, W3, b3):
    ei = edge_index.astype(jnp.int32)
    hh = _round_call(h)
    binned, counts = _prep(ei[0], ei[1])
    g = None
    deg2 = None
    for li, (W, b) in enumerate(((W1, b1), (W2, b2), (W3, b3))):
        if li == 0:
            ssum, ssq, smx, smn, deg = _agg_deg(hh, binned, counts)
            deg2 = deg.reshape(_NPAD, 16)
        else:
            ssum, ssq, smx, smn = _agg_nodeg(hh, binned, counts)
        hh, g = _dense_layer(ssum, ssq, smx, smn, deg2, W, b)
    return g
